# baseline (speedup 1.0000x reference)
import functools

import jax
import jax.numpy as jnp
from jax import lax
from jax.experimental import pallas as pl
from jax.experimental.pallas import tpu as pltpu
from jax.experimental.pallas import tpu_sc as plsc

_CHUNK = 128
_L = 16


@functools.lru_cache(maxsize=None)
def _build(B, V, D):
    info = plsc.get_sparse_core_info()
    NC, NS = info.num_cores, info.num_subcores
    NW = NC * NS
    assert B % (NW * _CHUNK) == 0, (B, NW)
    b_per_w = B // NW
    n_words = b_per_w * D
    n_streams = n_words // _CHUNK

    mesh = plsc.VectorSubcoreMesh(core_axis_name="c", subcore_axis_name="s")

    @functools.partial(
        pl.kernel,
        mesh=mesh,
        compiler_params=pltpu.CompilerParams(
            use_tc_tiling_on_sc=False, needs_layout_passes=False
        ),
        out_type=jax.ShapeDtypeStruct((B * D // _CHUNK, _CHUNK), jnp.float32),
        scratch_types=[
            pltpu.VMEM((b_per_w,), jnp.int32),
            pltpu.VMEM((n_streams, _CHUNK), jnp.int32),
            pltpu.VMEM((n_streams, _CHUNK), jnp.float32),
            pltpu.SemaphoreType.DMA,
        ],
    )
    def gather_kernel(ids_hbm, tableT_hbm, out_hbm, idx_v, pidx_v, out_v, sem):
        wid = lax.axis_index("s") * NC + lax.axis_index("c")
        base = wid * b_per_w
        pltpu.sync_copy(ids_hbm.at[pl.ds(base, b_per_w)], idx_v)
        lanes = lax.iota(jnp.int32, _L)

        def compute_block(g, carry):
            id16 = idx_v[pl.ds(g * _L, _L)]
            k_base = (g * _L + lanes) * D
            for c in range(D):
                pidx = id16 + c * V
                k = k_base + c
                plsc.store_scatter(
                    pidx_v,
                    [lax.shift_right_logical(k, 7), lax.bitwise_and(k, 127)],
                    pidx,
                )
            return carry

        lax.fori_loop(0, b_per_w // _L, compute_block, 0)

        flat_table = tableT_hbm.at[0]
        copies = []
        for q in range(n_streams):
            copies.append(
                pltpu.async_copy(flat_table.at[pidx_v.at[q]], out_v.at[q], sem)
            )
        for cp in copies:
            cp.wait()
        pltpu.sync_copy(out_v, out_hbm.at[pl.ds(wid * n_streams, n_streams)])

    return gather_kernel


def kernel(phoneme_ids, table):
    (B,) = phoneme_ids.shape
    V, D = table.shape
    fn = _build(B, V, D)
    tableT = jnp.swapaxes(table, 0, 1)
    out = fn(phoneme_ids.astype(jnp.int32), tableT)
    return out.reshape(B, D)


# speedup vs baseline: 4.7046x; 4.7046x over previous
import functools

import jax
import jax.numpy as jnp
from jax import lax
from jax.experimental import pallas as pl
from jax.experimental.pallas import tpu as pltpu
from jax.experimental.pallas import tpu_sc as plsc

_CHUNK = 128
_L = 16


@functools.lru_cache(maxsize=None)
def _build(B, V, D):
    info = plsc.get_sparse_core_info()
    NC, NS = info.num_cores, info.num_subcores
    NW = NC * NS
    assert B % (NW * _CHUNK) == 0, (B, NW)
    b_per_w = B // NW
    n_words = b_per_w * D
    n_streams = n_words // _CHUNK

    mesh = plsc.VectorSubcoreMesh(core_axis_name="c", subcore_axis_name="s")

    @functools.partial(
        pl.kernel,
        mesh=mesh,
        compiler_params=pltpu.CompilerParams(
            use_tc_tiling_on_sc=False, needs_layout_passes=False
        ),
        out_type=jax.ShapeDtypeStruct((B * D // _CHUNK, _CHUNK), jnp.float32),
        scratch_types=[
            pltpu.VMEM((b_per_w,), jnp.int32),
            pltpu.VMEM((n_streams, _CHUNK), jnp.int32),
            pltpu.VMEM((n_streams, _CHUNK), jnp.float32),
            pltpu.SemaphoreType.DMA,
        ],
    )
    def gather_kernel(ids_hbm, tableT_hbm, out_hbm, idx_v, pidx_v, out_v, sem):
        wid = lax.axis_index("s") * NC + lax.axis_index("c")
        base = wid * b_per_w
        pltpu.sync_copy(ids_hbm.at[pl.ds(base, b_per_w)], idx_v)
        lanes = lax.iota(jnp.int32, _L)

        def compute_block(g, carry):
            id16 = idx_v[pl.ds(g * _L, _L)] * D
            k_base = (g * _L + lanes) * D
            for c in range(D):
                pidx = id16 + c
                k = k_base + c
                plsc.store_scatter(
                    pidx_v,
                    [lax.shift_right_logical(k, 7), lax.bitwise_and(k, 127)],
                    pidx,
                )
            return carry

        lax.fori_loop(0, b_per_w // _L, compute_block, 0)

        flat_table = tableT_hbm.at[0]
        copies = []
        for q in range(n_streams):
            copies.append(
                pltpu.async_copy(flat_table.at[pidx_v.at[q]], out_v.at[q], sem)
            )
        for cp in copies:
            cp.wait()
        pltpu.sync_copy(out_v, out_hbm.at[pl.ds(wid * n_streams, n_streams)])

    return gather_kernel


def kernel(phoneme_ids, table):
    (B,) = phoneme_ids.shape
    V, D = table.shape
    fn = _build(B, V, D)
    table128 = table.reshape(V * D // _CHUNK, _CHUNK)
    out = fn(phoneme_ids.astype(jnp.int32), table128)
    return out.reshape(B, D)


# speedup vs baseline: 4.9764x; 1.0578x over previous
import functools

import jax
import jax.numpy as jnp
from jax import lax
from jax.experimental import pallas as pl
from jax.experimental.pallas import tpu as pltpu
from jax.experimental.pallas import tpu_sc as plsc

_CHUNK = 128


@functools.lru_cache(maxsize=None)
def _build(B, V, D):
    info = plsc.get_sparse_core_info()
    NC, NS = info.num_cores, info.num_subcores
    NW = NC * NS
    assert B % (NW * _CHUNK) == 0, (B, NW)
    b_per_w = B // NW
    n_chunks = b_per_w // _CHUNK
    row_group = _CHUNK // D

    mesh = plsc.VectorSubcoreMesh(core_axis_name="c", subcore_axis_name="s")

    @functools.partial(
        pl.kernel,
        mesh=mesh,
        compiler_params=pltpu.CompilerParams(use_tc_tiling_on_sc=False),
        out_type=jax.ShapeDtypeStruct((B // row_group, _CHUNK), jnp.float32),
        scratch_types=[
            pltpu.VMEM((n_chunks, _CHUNK), jnp.int32),
            pltpu.VMEM((n_chunks, _CHUNK, D), jnp.float32),
            pltpu.SemaphoreType.DMA,
        ],
    )
    def gather_kernel(ids_hbm, table_hbm, out_hbm, idx_v, rows_v, sem):
        wid = lax.axis_index("s") * NC + lax.axis_index("c")
        obase = wid * (b_per_w // row_group)
        pltpu.sync_copy(ids_hbm.at[wid], idx_v)
        copies = []
        for q in range(n_chunks):
            copies.append(
                pltpu.async_copy(
                    table_hbm.at[idx_v.at[q]], rows_v.at[q], sem
                )
            )
        for cp in copies:
            cp.wait()
        for q in range(n_chunks):
            pltpu.sync_copy(
                rows_v.at[q],
                out_hbm.at[
                    pl.ds(obase, b_per_w // row_group), pl.ds(q * D, D)
                ],
            )

    return gather_kernel


def kernel(phoneme_ids, table):
    (B,) = phoneme_ids.shape
    V, D = table.shape
    fn = _build(B, V, D)
    info = plsc.get_sparse_core_info()
    NW = info.num_cores * info.num_subcores
    n_chunks = B // (NW * _CHUNK)
    ids_r = (
        phoneme_ids.astype(jnp.int32)
        .reshape(NW, _CHUNK, n_chunks)
        .transpose(0, 2, 1)
    )
    out = fn(ids_r, table)
    return out.reshape(B, D)
